# baseline (speedup 1.0000x reference)
import dataclasses

import jax
import jax.numpy as jnp
from jax.experimental import pallas as pl
from jax.experimental.pallas import tpu as pltpu
from jax.experimental.pallas import tpu_sc as plsc

D = 1024
F = 2048
E = 8
K = 2
EPS = 1e-06
T = 256
RS = D ** -0.5
NF = 2
FB = F // NF
NF2 = 4
FB2 = F // NF2
NLANE = 16
NCHUNK = T // NLANE


def _split(x):
    hi = x.astype(jnp.bfloat16)
    lo = (x - hi.astype(jnp.float32)).astype(jnp.bfloat16)
    return hi, lo


def _mm3(x, w):
    xh, xl = _split(x)
    wh, wl = _split(w)
    o = jnp.dot(xh, wh, preferred_element_type=jnp.float32)
    o += jnp.dot(xh, wl, preferred_element_type=jnp.float32)
    o += jnp.dot(xl, wh, preferred_element_type=jnp.float32)
    return o


def _mm1(x, w):
    return jnp.dot(x, w, preferred_element_type=jnp.float32,
                   precision=jax.lax.Precision.DEFAULT)


def _prologue_kernel(x_ref, gk_ref, pln2_ref, pfs2_ref,
                     xr_ref, ltT_ref, meanp_ref):
    x = x_ref[...]
    var = jnp.mean(x * x, axis=-1, keepdims=True)
    inv = jax.lax.rsqrt(var + EPS)
    xn = x * inv
    xr_ref[...] = xn * pln2_ref[...]
    gate_in = xn * RS * pfs2_ref[...]
    logits = jnp.dot(gate_in.astype(jnp.bfloat16),
                     gk_ref[...].astype(jnp.bfloat16),
                     preferred_element_type=jnp.float32)
    ltT_ref[...] = logits
    m1 = jnp.max(logits, axis=1, keepdims=True)
    ex = jnp.exp(logits - m1)
    probs = ex / jnp.sum(ex, axis=1, keepdims=True)
    meanp_ref[0, :] = jnp.mean(probs, axis=0)


def _router_sc(lt_hbm, comb_hbm, oh_hbm, lt_v, comb_v, oh_v):
    c = jax.lax.axis_index("c")
    s = jax.lax.axis_index("s")
    u = c * 16 + s

    @pl.when(u < NCHUNK)
    def _():
        base = u * NLANE * E
        pltpu.sync_copy(lt_hbm.at[pl.ds(base, NLANE * E)], lt_v)
        iota = jax.lax.iota(jnp.int32, NLANE)
        idx = [iota * E + e for e in range(E)]
        l = [plsc.load_gather(lt_v, [idx[e]]) for e in range(E)]
        m1 = l[0]
        i1 = jnp.zeros((NLANE,), jnp.int32)
        for e in range(1, E):
            gt = l[e] > m1
            m1 = jnp.where(gt, l[e], m1)
            i1 = jnp.where(gt, e, i1)
        m2 = jnp.full((NLANE,), -jnp.inf, jnp.float32)
        i2 = jnp.full((NLANE,), E, jnp.int32)
        for e in range(E):
            gt2 = jnp.logical_and(l[e] > m2, i1 != e)
            m2 = jnp.where(gt2, l[e], m2)
            i2 = jnp.where(gt2, e, i2)
        b = jnp.exp(m2 - m1)
        p2 = b / (1.0 + b)
        p1 = 1.0 - p2
        zero = jnp.zeros((NLANE,), jnp.float32)
        one = jnp.full((NLANE,), 1.0, jnp.float32)
        for e in range(E):
            s1 = i1 == e
            s2 = i2 == e
            cv = jnp.where(s1, p1, zero) + jnp.where(s2, p2, zero)
            ov = jnp.where(jnp.logical_or(s1, s2), one, zero)
            plsc.store_scatter(comb_v, [idx[e]], cv)
            plsc.store_scatter(oh_v, [idx[e]], ov)
        pltpu.sync_copy(comb_v, comb_hbm.at[pl.ds(base, NLANE * E)])
        pltpu.sync_copy(oh_v, oh_hbm.at[pl.ds(base, NLANE * E)])


def _router(lt_flat):
    mesh = plsc.VectorSubcoreMesh(core_axis_name="c", subcore_axis_name="s")
    cp = pltpu.CompilerParams()
    if "needs_layout_passes" in pltpu.CompilerParams.__dataclass_fields__:
        cp = dataclasses.replace(cp, needs_layout_passes=False)
    f = pl.kernel(
        _router_sc,
        compiler_params=cp,
        out_type=[
            jax.ShapeDtypeStruct((T * E,), jnp.float32),
            jax.ShapeDtypeStruct((T * E,), jnp.float32),
        ],
        mesh=mesh,
        scratch_types=[
            pltpu.VMEM((NLANE * E,), jnp.float32),
            pltpu.VMEM((NLANE * E,), jnp.float32),
            pltpu.VMEM((NLANE * E,), jnp.float32),
        ],
    )
    return f(lt_flat)


def _main_kernel(x_ref, xr_ref, combT_ref, ohT_ref, meanp_ref,
                 pln1_ref, plnr_ref, swi0_ref, swi1_ref, swo_ref,
                 wi0_ref, wi1_ref, wo_ref,
                 out_ref, stats_ref, sh_s, racc_s):
    e = pl.program_id(0)
    j = pl.program_id(1)
    flat = e * NF + j

    @pl.when(flat == 0)
    def _init():
        sh_s[...] = jnp.zeros_like(sh_s)
        racc_s[...] = jnp.zeros_like(racc_s)

    @pl.when(flat < NF2)
    def _shared_chunk():
        x = x_ref[...]
        h0 = _mm1(x, swi0_ref[...])
        h1 = _mm1(x, swi1_ref[...])
        sh_s[...] += _mm1(jax.nn.gelu(h0) * h1, swo_ref[...])

    xr = xr_ref[...]
    g0 = _mm1(xr, wi0_ref[0])
    g1 = _mm1(xr, wi1_ref[0])
    h = jax.nn.gelu(g0) * g1
    iota = jax.lax.broadcasted_iota(jnp.int32, (T, E), 1)
    w = jnp.sum(jnp.where(iota == e, combT_ref[...], 0.0), axis=1,
                keepdims=True)
    racc_s[...] += _mm1(h, wo_ref[0]) * w

    @pl.when(flat == E * NF - 1)
    def _finalize():
        s = sh_s[...]
        svar = jnp.mean(s * s, axis=-1, keepdims=True)
        sn = s * jax.lax.rsqrt(svar + EPS) * pln1_ref[...]
        r = racc_s[...]
        rvar = jnp.mean(r * r, axis=-1, keepdims=True)
        rn = r * jax.lax.rsqrt(rvar + EPS) * plnr_ref[...]
        out_ref[...] = rn + sn
        counts = jnp.sum(ohT_ref[...], axis=0)
        df = counts / (T * K)
        lbl = E * jnp.sum(df * meanp_ref[0, :])
        stats_ref[0, :] = df - 1.0 / E
        stats_ref[1, :] = jnp.full((E,), lbl, jnp.float32)


def kernel(inputs, pre_forward_scale_2, pre_ln2_scale, post_ln1_scale,
           post_ln2_scale, gate_kernel, shared_wi_0, shared_wi_1, shared_wo,
           routed_wi_0, routed_wi_1, routed_wo):
    x = inputs.reshape(T, D)
    pln2 = pre_ln2_scale.reshape(1, D)
    pfs2 = pre_forward_scale_2.reshape(1, D)
    pln1 = post_ln1_scale.reshape(1, D)
    plnr = post_ln2_scale.reshape(1, D)

    xr, ltT, meanp = pl.pallas_call(
        _prologue_kernel,
        out_shape=[
            jax.ShapeDtypeStruct((T, D), jnp.float32),
            jax.ShapeDtypeStruct((T, E), jnp.float32),
            jax.ShapeDtypeStruct((1, E), jnp.float32),
        ],
    )(x, gate_kernel, pln2, pfs2)

    combF, ohF = _router(ltT.reshape(T * E))
    combT = combF.reshape(T, E)
    ohT = ohF.reshape(T, E)

    out, stats = pl.pallas_call(
        _main_kernel,
        grid=(E, NF),
        in_specs=[
            pl.BlockSpec((T, D), lambda e, j: (0, 0)),
            pl.BlockSpec((T, D), lambda e, j: (0, 0)),
            pl.BlockSpec((T, E), lambda e, j: (0, 0)),
            pl.BlockSpec((T, E), lambda e, j: (0, 0)),
            pl.BlockSpec((1, E), lambda e, j: (0, 0)),
            pl.BlockSpec((1, D), lambda e, j: (0, 0)),
            pl.BlockSpec((1, D), lambda e, j: (0, 0)),
            pl.BlockSpec((D, FB2),
                         lambda e, j: (0, jnp.minimum(e * NF + j, NF2 - 1))),
            pl.BlockSpec((D, FB2),
                         lambda e, j: (0, jnp.minimum(e * NF + j, NF2 - 1))),
            pl.BlockSpec((FB2, D),
                         lambda e, j: (jnp.minimum(e * NF + j, NF2 - 1), 0)),
            pl.BlockSpec((1, D, FB), lambda e, j: (e, 0, j)),
            pl.BlockSpec((1, D, FB), lambda e, j: (e, 0, j)),
            pl.BlockSpec((1, FB, D), lambda e, j: (e, j, 0)),
        ],
        out_specs=[
            pl.BlockSpec((T, D), lambda e, j: (0, 0)),
            pl.BlockSpec((2, E), lambda e, j: (0, 0)),
        ],
        out_shape=[
            jax.ShapeDtypeStruct((T, D), jnp.float32),
            jax.ShapeDtypeStruct((2, E), jnp.float32),
        ],
        scratch_shapes=[
            pltpu.VMEM((T, D), jnp.float32),
            pltpu.VMEM((T, D), jnp.float32),
        ],
    )(x, xr, combT, ohT, meanp, pln1, plnr,
      shared_wi_0, shared_wi_1, shared_wo,
      routed_wi_0, routed_wi_1, routed_wo)

    return out.reshape(inputs.shape), stats[1, 0], stats[0]


# speedup vs baseline: 1.2801x; 1.2801x over previous
import jax
import jax.numpy as jnp
from jax.experimental import pallas as pl
from jax.experimental.pallas import tpu as pltpu

D = 1024
F = 2048
E = 8
K = 2
EPS = 1e-06
T = 256
RS = D ** -0.5
NF = 2
FB = F // NF
NF2 = 4
FB2 = F // NF2


def _split(x):
    hi = x.astype(jnp.bfloat16)
    lo = (x - hi.astype(jnp.float32)).astype(jnp.bfloat16)
    return hi, lo


def _mm3(x, w):
    xh, xl = _split(x)
    wh, wl = _split(w)
    o = jnp.dot(xh, wh, preferred_element_type=jnp.float32)
    o += jnp.dot(xh, wl, preferred_element_type=jnp.float32)
    o += jnp.dot(xl, wh, preferred_element_type=jnp.float32)
    return o


def _mm1(x, w):
    return jnp.dot(x, w, preferred_element_type=jnp.float32,
                   precision=jax.lax.Precision.DEFAULT)


def _moe_kernel(x_ref, gk_ref, pln2_ref, pfs2_ref, pln1_ref, plnr_ref,
                swi0_ref, swi1_ref, swo_ref, wi0_ref, wi1_ref, wo_ref,
                out_ref, stats_ref,
                xr_s, comb_s, sh_s, racc_s):
    e = pl.program_id(0)
    j = pl.program_id(1)
    flat = e * NF + j

    @pl.when(flat == 0)
    def _prologue():
        x = x_ref[...]
        var = jnp.mean(x * x, axis=-1, keepdims=True)
        inv = jax.lax.rsqrt(var + EPS)
        xn = x * inv
        xr_s[...] = xn * pln2_ref[...]
        gate_in = xn * RS * pfs2_ref[...]
        logits = jnp.dot(gate_in.astype(jnp.bfloat16),
                         gk_ref[...].astype(jnp.bfloat16),
                         preferred_element_type=jnp.float32)

        iota = jax.lax.broadcasted_iota(jnp.int32, (T, E), 1)
        m1 = jnp.max(logits, axis=1, keepdims=True)
        i1 = jnp.min(jnp.where(logits == m1, iota, E), axis=1, keepdims=True)
        lg2 = jnp.where(iota == i1, -jnp.inf, logits)
        m2 = jnp.max(lg2, axis=1, keepdims=True)
        i2 = jnp.min(jnp.where(lg2 == m2, iota, E), axis=1, keepdims=True)
        b = jnp.exp(m2 - m1)
        p1 = 1.0 / (1.0 + b)
        p2 = 1.0 - p1
        oh1 = (iota == i1).astype(jnp.float32)
        oh2 = (iota == i2).astype(jnp.float32)
        comb_s[...] = oh1 * p1 + oh2 * p2

        ex = jnp.exp(logits - m1)
        probs = ex / jnp.sum(ex, axis=1, keepdims=True)
        mean_probs = jnp.mean(probs, axis=0)
        counts = jnp.sum(oh1 + oh2, axis=0)
        df = counts / (T * K)
        lbl = E * jnp.sum(df * mean_probs)
        stats_ref[0, :] = df - 1.0 / E
        stats_ref[1, :] = jnp.full((E,), lbl, jnp.float32)

        sh_s[...] = jnp.zeros_like(sh_s)
        racc_s[...] = jnp.zeros_like(racc_s)

    @pl.when(flat < NF2)
    def _shared_chunk():
        x = x_ref[...]
        h0 = _mm1(x, swi0_ref[...])
        h1 = _mm1(x, swi1_ref[...])
        sh_s[...] += _mm1(jax.nn.gelu(h0) * h1, swo_ref[...])

    xr = xr_s[...]
    g0 = _mm1(xr, wi0_ref[0])
    g1 = _mm1(xr, wi1_ref[0])
    h = jax.nn.gelu(g0) * g1
    iota = jax.lax.broadcasted_iota(jnp.int32, (T, E), 1)
    w = jnp.sum(jnp.where(iota == e, comb_s[...], 0.0), axis=1, keepdims=True)
    racc_s[...] += _mm1(h, wo_ref[0]) * w

    @pl.when(flat == E * NF - 1)
    def _finalize():
        s = sh_s[...]
        svar = jnp.mean(s * s, axis=-1, keepdims=True)
        sn = s * jax.lax.rsqrt(svar + EPS) * pln1_ref[...]
        r = racc_s[...]
        rvar = jnp.mean(r * r, axis=-1, keepdims=True)
        rn = r * jax.lax.rsqrt(rvar + EPS) * plnr_ref[...]
        out_ref[...] = rn + sn


def kernel(inputs, pre_forward_scale_2, pre_ln2_scale, post_ln1_scale,
           post_ln2_scale, gate_kernel, shared_wi_0, shared_wi_1, shared_wo,
           routed_wi_0, routed_wi_1, routed_wo):
    x = inputs.reshape(T, D)
    pln2 = pre_ln2_scale.reshape(1, D)
    pfs2 = pre_forward_scale_2.reshape(1, D)
    pln1 = post_ln1_scale.reshape(1, D)
    plnr = post_ln2_scale.reshape(1, D)

    out, stats = pl.pallas_call(
        _moe_kernel,
        grid=(E, NF),
        in_specs=[
            pl.BlockSpec((T, D), lambda e, j: (0, 0)),
            pl.BlockSpec((D, E), lambda e, j: (0, 0)),
            pl.BlockSpec((1, D), lambda e, j: (0, 0)),
            pl.BlockSpec((1, D), lambda e, j: (0, 0)),
            pl.BlockSpec((1, D), lambda e, j: (0, 0)),
            pl.BlockSpec((1, D), lambda e, j: (0, 0)),
            pl.BlockSpec((D, FB2),
                         lambda e, j: (0, jnp.minimum(e * NF + j, NF2 - 1))),
            pl.BlockSpec((D, FB2),
                         lambda e, j: (0, jnp.minimum(e * NF + j, NF2 - 1))),
            pl.BlockSpec((FB2, D),
                         lambda e, j: (jnp.minimum(e * NF + j, NF2 - 1), 0)),
            pl.BlockSpec((1, D, FB), lambda e, j: (e, 0, j)),
            pl.BlockSpec((1, D, FB), lambda e, j: (e, 0, j)),
            pl.BlockSpec((1, FB, D), lambda e, j: (e, j, 0)),
        ],
        out_specs=[
            pl.BlockSpec((T, D), lambda e, j: (0, 0)),
            pl.BlockSpec((2, E), lambda e, j: (0, 0)),
        ],
        out_shape=[
            jax.ShapeDtypeStruct((T, D), jnp.float32),
            jax.ShapeDtypeStruct((2, E), jnp.float32),
        ],
        scratch_shapes=[
            pltpu.VMEM((T, D), jnp.float32),
            pltpu.VMEM((T, E), jnp.float32),
            pltpu.VMEM((T, D), jnp.float32),
            pltpu.VMEM((T, D), jnp.float32),
        ],
    )(x, gate_kernel, pln2, pfs2, pln1, plnr,
      shared_wi_0, shared_wi_1, shared_wo,
      routed_wi_0, routed_wi_1, routed_wo)

    return out.reshape(inputs.shape), stats[1, 0], stats[0]
